# Initial kernel scaffold; baseline (speedup 1.0000x reference)
import jax
import jax.numpy as jnp
from jax.experimental import pallas as pl


def kernel(source, target):
    raise NotImplementedError("write your pallas kernel here")


# speedup vs baseline: 1.3641x; 1.3641x over previous
import jax
import jax.numpy as jnp
from jax.experimental import pallas as pl


def _cd_pair_kernel(s_ref, t_ref, out_ref):
    s = s_ref[0]
    t = t_ref[0]
    ns = jnp.sum(s * s, axis=1)
    nt = jnp.sum(t * t, axis=1)
    cross = jax.lax.dot_general(
        -2.0 * s, t, (((1,), (1,)), ((), ())), preferred_element_type=jnp.float32
    )
    d2 = (cross + ns[:, None]) + nt[None, :]
    rowmin = jnp.min(d2, axis=1)
    colmin = jnp.min(d2, axis=0)
    fwd = jnp.mean(jnp.sqrt(jnp.maximum(rowmin, 1e-12)))
    bwd = jnp.mean(jnp.sqrt(jnp.maximum(colmin, 1e-12)))
    out_ref[...] = jnp.full((1, 1, 1, 128), fwd + bwd, dtype=jnp.float32)


@jax.jit
def kernel(source, target):
    b, n, d = source.shape
    cd = pl.pallas_call(
        _cd_pair_kernel,
        grid=(b, b),
        in_specs=[
            pl.BlockSpec((1, n, d), lambda i, j: (i, 0, 0)),
            pl.BlockSpec((1, n, d), lambda i, j: (j, 0, 0)),
        ],
        out_specs=pl.BlockSpec((1, 1, 1, 128), lambda i, j: (i, j, 0, 0)),
        out_shape=jax.ShapeDtypeStruct((b, b, 1, 128), jnp.float32),
    )(source, target)[:, :, 0, 0]

    return jnp.mean(jnp.min(cd, axis=1))


# speedup vs baseline: 1.5236x; 1.1169x over previous
import jax
import jax.numpy as jnp
from jax.experimental import pallas as pl


def _cd_pair_kernel(s_ref, t_ref, out_ref):
    s = s_ref[0]
    t = t_ref[0]
    d2 = jax.lax.dot_general(
        s, t, (((1,), (1,)), ((), ())), preferred_element_type=jnp.float32
    )
    rowmin = jnp.min(d2, axis=1)
    colmin = jnp.min(d2, axis=0)
    fwd = jnp.mean(jnp.sqrt(jnp.maximum(rowmin, 1e-12)))
    bwd = jnp.mean(jnp.sqrt(jnp.maximum(colmin, 1e-12)))
    out_ref[...] = jnp.full((1, 1, 1, 128), fwd + bwd, dtype=jnp.float32)


def _hi_lo(x):
    hi = x.astype(jnp.bfloat16).astype(jnp.float32)
    return hi, x - hi


@jax.jit
def kernel(source, target):
    b, n, _ = source.shape
    ns = jnp.sum(source * source, axis=-1, keepdims=True)
    nt = jnp.sum(target * target, axis=-1, keepdims=True)
    ns_hi, ns_lo = _hi_lo(ns)
    nt_hi, nt_lo = _hi_lo(nt)
    s_r = source.astype(jnp.bfloat16).astype(jnp.float32)
    t_r = target.astype(jnp.bfloat16).astype(jnp.float32)
    ones = jnp.ones_like(ns)
    zero = jnp.zeros_like(ns)
    s_aug = jnp.concatenate([-2.0 * s_r, ns_hi, ns_lo, ones, ones, zero], axis=-1)
    t_aug = jnp.concatenate([t_r, ones, ones, nt_hi, nt_lo, zero], axis=-1)

    cd = pl.pallas_call(
        _cd_pair_kernel,
        grid=(b, b),
        in_specs=[
            pl.BlockSpec((1, n, 8), lambda i, j: (i, 0, 0)),
            pl.BlockSpec((1, n, 8), lambda i, j: (j, 0, 0)),
        ],
        out_specs=pl.BlockSpec((1, 1, 1, 128), lambda i, j: (i, j, 0, 0)),
        out_shape=jax.ShapeDtypeStruct((b, b, 1, 128), jnp.float32),
    )(s_aug, t_aug)[:, :, 0, 0]

    return jnp.mean(jnp.min(cd, axis=1))


# speedup vs baseline: 1.5462x; 1.0148x over previous
import jax
import jax.numpy as jnp
from jax.experimental import pallas as pl


def _cd_pair_kernel(s_ref, t_ref, out_ref):
    s = s_ref[0]
    t = t_ref[0]
    d2 = jax.lax.dot_general(
        s, t, (((1,), (1,)), ((), ())), preferred_element_type=jnp.float32
    )
    rowmin = jnp.min(d2, axis=1)
    colmin = jnp.min(d2, axis=0)
    fwd = jnp.mean(jnp.sqrt(jnp.maximum(rowmin, 1e-12)))
    bwd = jnp.mean(jnp.sqrt(jnp.maximum(colmin, 1e-12)))
    out_ref[...] = jnp.full((1, 1, 1, 128), fwd + bwd, dtype=jnp.float32)


def _hi_lo(x):
    hi = x.astype(jnp.bfloat16).astype(jnp.float32)
    return hi, x - hi


@jax.jit
def kernel(source, target):
    b, n, _ = source.shape
    ns = jnp.sum(source * source, axis=-1, keepdims=True)
    nt = jnp.sum(target * target, axis=-1, keepdims=True)
    ns_hi, ns_lo = _hi_lo(ns)
    nt_hi, nt_lo = _hi_lo(nt)
    s_r = source.astype(jnp.bfloat16).astype(jnp.float32)
    t_r = target.astype(jnp.bfloat16).astype(jnp.float32)
    ones = jnp.ones_like(ns)
    zero = jnp.zeros_like(ns)
    s_aug = jnp.concatenate([-2.0 * s_r, ns_hi, ns_lo, ones, ones, zero], axis=-1)
    t_aug = jnp.concatenate([t_r, ones, ones, nt_hi, nt_lo, zero], axis=-1)
    s_aug = s_aug.astype(jnp.bfloat16)
    t_aug = t_aug.astype(jnp.bfloat16)

    cd = pl.pallas_call(
        _cd_pair_kernel,
        grid=(b, b),
        in_specs=[
            pl.BlockSpec((1, n, 8), lambda i, j: (i, 0, 0)),
            pl.BlockSpec((1, n, 8), lambda i, j: (j, 0, 0)),
        ],
        out_specs=pl.BlockSpec((1, 1, 1, 128), lambda i, j: (i, j, 0, 0)),
        out_shape=jax.ShapeDtypeStruct((b, b, 1, 128), jnp.float32),
    )(s_aug, t_aug)[:, :, 0, 0]

    return jnp.mean(jnp.min(cd, axis=1))
